# baseline (speedup 1.0000x reference)
import functools

import jax
import jax.numpy as jnp
from jax import lax
from jax.experimental import pallas as pl
from jax.experimental.pallas import tpu as pltpu
from jax.experimental.pallas import tpu_sc as plsc

N = 10000
D = 128
E = 320000
DW = 128
NC = 2
NS = 16
NW = NC * NS
EPW = E // NW
K = 80
NCH = EPW // K
STRIPE = 624
TAIL = N - STRIPE * NS
QD = 8

_mesh = plsc.VectorSubcoreMesh(core_axis_name="c", subcore_axis_name="s")


def _zero_stripe(zeros_hbm, acc, sid):
    row0 = sid * STRIPE
    pltpu.sync_copy(zeros_hbm.at[pl.ds(row0, STRIPE)], acc.at[pl.ds(row0, STRIPE)])

    @pl.when(sid == NS - 1)
    def _():
        pltpu.sync_copy(zeros_hbm.at[pl.ds(STRIPE * NS, TAIL)],
                        acc.at[pl.ds(STRIPE * NS, TAIL)])


def _write_stripe(acc, out_hbm, cid, sid):
    row0 = sid * STRIPE
    pltpu.sync_copy(acc.at[pl.ds(row0, STRIPE)],
                    out_hbm.at[cid, pl.ds(row0, STRIPE)])

    @pl.when(sid == NS - 1)
    def _():
        pltpu.sync_copy(acc.at[pl.ds(STRIPE * NS, TAIL)],
                        out_hbm.at[cid, pl.ds(STRIPE * NS, TAIL)])


def _sc_degree(dst3, ones, zeros_w):
    @functools.partial(
        pl.kernel,
        out_type=jax.ShapeDtypeStruct((NC, N, DW), jnp.float32),
        mesh=_mesh,
        scratch_types=[
            pltpu.VMEM((NCH, K), jnp.int32),
            pltpu.VMEM((K, DW), jnp.float32),
            pltpu.VMEM_SHARED((N, DW), jnp.float32),
            pltpu.SemaphoreType.DMA,
        ],
    )
    def k(dst_hbm, ones_hbm, zeros_hbm, out_hbm, didx, ones_v, dacc, sem):
        cid = lax.axis_index("c")
        sid = lax.axis_index("s")
        wid = sid * NC + cid
        pltpu.sync_copy(dst_hbm.at[wid], didx)
        pltpu.sync_copy(ones_hbm, ones_v)
        _zero_stripe(zeros_hbm, dacc, sid)
        plsc.subcore_barrier()

        def scat(j):
            pltpu.async_copy(ones_v, dacc.at[didx.at[j]], sem, add=True)

        def swait():
            pltpu.make_async_copy(ones_v, dacc.at[didx.at[0]], sem).wait()

        for j in range(QD):
            scat(j)

        def body(j, carry):
            swait()
            scat(j)
            return carry

        lax.fori_loop(QD, NCH, body, 0)
        for _ in range(QD):
            swait()
        plsc.subcore_barrier()
        _write_stripe(dacc, out_hbm, cid, sid)

    return k(dst3, ones, zeros_w)


def _sc_edge_aggregate(hs, sd4, zeros_d):
    @functools.partial(
        pl.kernel,
        out_type=jax.ShapeDtypeStruct((NC, N, D), jnp.float32),
        mesh=_mesh,
        scratch_types=[
            pltpu.VMEM((8, 2, K), jnp.int32),
            pltpu.VMEM((4, K, D), jnp.float32),
            pltpu.VMEM_SHARED((N, D), jnp.float32),
            [pltpu.SemaphoreType.DMA for _ in range(4)],
            [pltpu.SemaphoreType.DMA for _ in range(4)],
            [pltpu.SemaphoreType.DMA for _ in range(8)],
        ],
    )
    def k(hs_hbm, sd_hbm, zeros_hbm, out_hbm,
          sdbuf, rows, acc, gsem, ssem, isem):
        cid = lax.axis_index("c")
        sid = lax.axis_index("s")
        wid = sid * NC + cid

        def gather(j, b, d):
            pltpu.async_copy(hs_hbm.at[sdbuf.at[d, 0]], rows.at[b], gsem[b])

        def gwait(j, b, d):
            pltpu.make_async_copy(hs_hbm.at[sdbuf.at[d, 0]], rows.at[b],
                                  gsem[b]).wait()

        def scat(j, b, d):
            pltpu.async_copy(rows.at[b], acc.at[sdbuf.at[d, 1]], ssem[b],
                             add=True)

        def swait(b, d):
            pltpu.make_async_copy(rows.at[b], acc.at[sdbuf.at[d, 1]],
                                  ssem[b]).wait()

        def icopy(j, d):
            pltpu.async_copy(sd_hbm.at[wid, j], sdbuf.at[d], isem[d])

        def iwait(j, d):
            pltpu.make_async_copy(sd_hbm.at[wid, 0], sdbuf.at[d],
                                  isem[d]).wait()

        def when_(cond, fn):
            if isinstance(cond, bool):
                if cond:
                    fn()
            else:
                pl.when(cond)(fn)

        icopy(0, 0)
        icopy(1, 1)
        iwait(0, 0)
        gather(0, 0, 0)
        _zero_stripe(zeros_hbm, acc, sid)
        plsc.subcore_barrier()

        def step(j, m):
            b = m % 4
            gwait(j, b, m)
            scat(j, b, m)

            def _next():
                when_(j >= 3,
                      lambda: swait((m + 1) % 4, (m + 5) % 8))
                iwait(j + 1, (m + 1) % 8)
                gather(j + 1, (b + 1) % 4, (m + 1) % 8)

            when_(j + 1 < NCH, _next)
            when_(j + 2 < NCH, lambda: icopy(j + 2, (m + 2) % 8))

        step(0, 0)
        step(1, 1)

        def oct_(t, carry):
            j0 = 8 * t + 2
            for i in range(8):
                step(j0 + i, (2 + i) % 8)
            return carry

        n_oct = (NCH - 2 - 3) // 8
        lax.fori_loop(0, n_oct, oct_, 0)
        for j in range(2 + 8 * n_oct, NCH):
            step(j, j % 8)
        swait((NCH - 4) % 4, (NCH - 4) % 8)
        swait((NCH - 3) % 4, (NCH - 3) % 8)
        swait((NCH - 2) % 4, (NCH - 2) % 8)
        swait((NCH - 1) % 4, (NCH - 1) % 8)
        plsc.subcore_barrier()
        _write_stripe(acc, out_hbm, cid, sid)

    return k(hs, sd4, zeros_d)


_RB = 5000


def _tc_proj(x, W, b):
    def body(x_ref, w_ref, b_ref, o_ref):
        h = jnp.dot(x_ref[...], w_ref[...], preferred_element_type=jnp.float32,
                    precision=lax.Precision.HIGHEST)
        o_ref[...] = jnp.maximum(h + b_ref[...][None, :], 0.0)

    return pl.pallas_call(
        body,
        grid=(N // _RB,),
        in_specs=[
            pl.BlockSpec((_RB, D), lambda i: (i, 0)),
            pl.BlockSpec((D, D), lambda i: (0, 0)),
            pl.BlockSpec((D,), lambda i: (0,)),
        ],
        out_specs=pl.BlockSpec((_RB, D), lambda i: (i, 0)),
        out_shape=jax.ShapeDtypeStruct((N, D), jnp.float32),
    )(x, W, b)


def _tc_hs0(h, W, dacc):
    def body(h_ref, w_ref, dacc_ref, hs_ref, dis_ref):
        deg = 1.0 + (dacc_ref[0, :, 0:1] + dacc_ref[1, :, 0:1])
        dis = lax.rsqrt(deg)
        dis_ref[...] = dis
        hl = jnp.dot(h_ref[...], w_ref[...], preferred_element_type=jnp.float32,
                     precision=lax.Precision.HIGHEST)
        hs_ref[...] = hl * dis

    return pl.pallas_call(
        body,
        grid=(N // _RB,),
        in_specs=[
            pl.BlockSpec((_RB, D), lambda i: (i, 0)),
            pl.BlockSpec((D, D), lambda i: (0, 0)),
            pl.BlockSpec((NC, _RB, DW), lambda i: (0, i, 0)),
        ],
        out_specs=[
            pl.BlockSpec((_RB, D), lambda i: (i, 0)),
            pl.BlockSpec((_RB, 1), lambda i: (i, 0)),
        ],
        out_shape=[
            jax.ShapeDtypeStruct((N, D), jnp.float32),
            jax.ShapeDtypeStruct((N, 1), jnp.float32),
        ],
    )(h, W, dacc)


def _tc_mid(h, hs, acc, dis, b, W_next):
    def body(h_ref, hs_ref, acc_ref, dis_ref, b_ref, w_ref, hn_ref, hsn_ref):
        dis = dis_ref[...]
        t = (acc_ref[0] + acc_ref[1] + hs_ref[...]) * dis
        h_new = h_ref[...] + jnp.maximum(t + b_ref[...][None, :], 0.0)
        hn_ref[...] = h_new
        hl = jnp.dot(h_new, w_ref[...], preferred_element_type=jnp.float32,
                     precision=lax.Precision.HIGHEST)
        hsn_ref[...] = hl * dis

    return pl.pallas_call(
        body,
        grid=(N // _RB,),
        in_specs=[
            pl.BlockSpec((_RB, D), lambda i: (i, 0)),
            pl.BlockSpec((_RB, D), lambda i: (i, 0)),
            pl.BlockSpec((NC, _RB, D), lambda i: (0, i, 0)),
            pl.BlockSpec((_RB, 1), lambda i: (i, 0)),
            pl.BlockSpec((D,), lambda i: (0,)),
            pl.BlockSpec((D, D), lambda i: (0, 0)),
        ],
        out_specs=[
            pl.BlockSpec((_RB, D), lambda i: (i, 0)),
            pl.BlockSpec((_RB, D), lambda i: (i, 0)),
        ],
        out_shape=[
            jax.ShapeDtypeStruct((N, D), jnp.float32),
            jax.ShapeDtypeStruct((N, D), jnp.float32),
        ],
    )(h, hs, acc, dis, b, W_next)


def _tc_fin(h, hs, acc, dis, b, W_out, b_out):
    def body(h_ref, hs_ref, acc_ref, dis_ref, b_ref, w_ref, bo_ref, o_ref):
        dis = dis_ref[...]
        t = (acc_ref[0] + acc_ref[1] + hs_ref[...]) * dis
        h_new = h_ref[...] + jnp.maximum(t + b_ref[...][None, :], 0.0)
        o = jnp.dot(h_new, w_ref[...], preferred_element_type=jnp.float32,
                    precision=lax.Precision.HIGHEST)
        o_ref[...] = o + bo_ref[...][None, :]

    return pl.pallas_call(
        body,
        grid=(N // _RB,),
        in_specs=[
            pl.BlockSpec((_RB, D), lambda i: (i, 0)),
            pl.BlockSpec((_RB, D), lambda i: (i, 0)),
            pl.BlockSpec((NC, _RB, D), lambda i: (0, i, 0)),
            pl.BlockSpec((_RB, 1), lambda i: (i, 0)),
            pl.BlockSpec((D,), lambda i: (0,)),
            pl.BlockSpec((D, D), lambda i: (0, 0)),
            pl.BlockSpec((D,), lambda i: (0,)),
        ],
        out_specs=pl.BlockSpec((_RB, D), lambda i: (i, 0)),
        out_shape=jax.ShapeDtypeStruct((N, D), jnp.float32),
    )(h, hs, acc, dis, b, W_out, b_out)


def kernel(x, edge_index, W_in, b_in, W_g0, b_g0, W_g1, b_g1, W_g2, b_g2,
           W_out, b_out):
    ei = edge_index.astype(jnp.int32)
    src3 = ei[0].reshape(NW, NCH, K)
    dst3 = ei[1].reshape(NW, NCH, K)
    sd4 = jnp.stack([src3, dst3], axis=2)
    ones_w = jnp.ones((K, DW), jnp.float32)
    zeros_w = jnp.zeros((N, DW), jnp.float32)
    zeros_d = jnp.zeros((N, D), jnp.float32)

    dacc = _sc_degree(dst3, ones_w, zeros_w)
    h = _tc_proj(x, W_in, b_in)
    hs, dis = _tc_hs0(h, W_g0, dacc)

    acc = _sc_edge_aggregate(hs, sd4, zeros_d)
    h, hs = _tc_mid(h, hs, acc, dis, b_g0, W_g1)

    acc = _sc_edge_aggregate(hs, sd4, zeros_d)
    h, hs = _tc_mid(h, hs, acc, dis, b_g1, W_g2)

    acc = _sc_edge_aggregate(hs, sd4, zeros_d)
    return _tc_fin(h, hs, acc, dis, b_g2, W_out, b_out)


# speedup vs baseline: 1.0124x; 1.0124x over previous
import functools

import jax
import jax.numpy as jnp
from jax import lax
from jax.experimental import pallas as pl
from jax.experimental.pallas import tpu as pltpu
from jax.experimental.pallas import tpu_sc as plsc

N = 10000
D = 128
E = 320000
DW = 128
NC = 2
NS = 16
NW = NC * NS
EPW = E // NW
K = 80
NCH = EPW // K
STRIPE = 624
TAIL = N - STRIPE * NS
QD = 8

_mesh = plsc.VectorSubcoreMesh(core_axis_name="c", subcore_axis_name="s")


def _zero_stripe(zeros_hbm, acc, sid):
    row0 = sid * STRIPE
    pltpu.sync_copy(zeros_hbm.at[pl.ds(row0, STRIPE)], acc.at[pl.ds(row0, STRIPE)])

    @pl.when(sid == NS - 1)
    def _():
        pltpu.sync_copy(zeros_hbm.at[pl.ds(STRIPE * NS, TAIL)],
                        acc.at[pl.ds(STRIPE * NS, TAIL)])


def _write_stripe(acc, out_hbm, cid, sid):
    row0 = sid * STRIPE
    pltpu.sync_copy(acc.at[pl.ds(row0, STRIPE)],
                    out_hbm.at[cid, pl.ds(row0, STRIPE)])

    @pl.when(sid == NS - 1)
    def _():
        pltpu.sync_copy(acc.at[pl.ds(STRIPE * NS, TAIL)],
                        out_hbm.at[cid, pl.ds(STRIPE * NS, TAIL)])


def _sc_degree(dst3, ones, zeros_w):
    @functools.partial(
        pl.kernel,
        out_type=jax.ShapeDtypeStruct((NC, N, DW), jnp.float32),
        mesh=_mesh,
        scratch_types=[
            pltpu.VMEM((NCH, K), jnp.int32),
            pltpu.VMEM((K, DW), jnp.float32),
            pltpu.VMEM_SHARED((N, DW), jnp.float32),
            pltpu.SemaphoreType.DMA,
        ],
    )
    def k(dst_hbm, ones_hbm, zeros_hbm, out_hbm, didx, ones_v, dacc, sem):
        cid = lax.axis_index("c")
        sid = lax.axis_index("s")
        wid = sid * NC + cid
        pltpu.sync_copy(dst_hbm.at[wid], didx)
        pltpu.sync_copy(ones_hbm, ones_v)
        _zero_stripe(zeros_hbm, dacc, sid)
        plsc.subcore_barrier()

        def scat(j):
            pltpu.async_copy(ones_v, dacc.at[didx.at[j]], sem, add=True)

        def swait():
            pltpu.make_async_copy(ones_v, dacc.at[didx.at[0]], sem).wait()

        for j in range(QD):
            scat(j)

        def body(j, carry):
            swait()
            scat(j)
            return carry

        lax.fori_loop(QD, NCH, body, 0)
        for _ in range(QD):
            swait()
        plsc.subcore_barrier()
        _write_stripe(dacc, out_hbm, cid, sid)

    return k(dst3, ones, zeros_w)


def _sc_edge_aggregate(hs, sd4, zeros_d):
    @functools.partial(
        pl.kernel,
        out_type=jax.ShapeDtypeStruct((NC, N, D), jnp.float32),
        mesh=_mesh,
        scratch_types=[
            pltpu.VMEM((8, 2, K), jnp.int32),
            pltpu.VMEM((4, K, D), jnp.float32),
            pltpu.VMEM_SHARED((N, D), jnp.float32),
            [pltpu.SemaphoreType.DMA for _ in range(4)],
            [pltpu.SemaphoreType.DMA for _ in range(4)],
            [pltpu.SemaphoreType.DMA for _ in range(8)],
        ],
    )
    def k(hs_hbm, sd_hbm, zeros_hbm, out_hbm,
          sdbuf, rows, acc, gsem, ssem, isem):
        cid = lax.axis_index("c")
        sid = lax.axis_index("s")
        wid = sid * NC + cid

        def gather(j, b, d):
            pltpu.async_copy(hs_hbm.at[sdbuf.at[d, 0]], rows.at[b], gsem[b])

        def gwait(j, b, d):
            pltpu.make_async_copy(hs_hbm.at[sdbuf.at[d, 0]], rows.at[b],
                                  gsem[b]).wait()

        def scat(j, b, d):
            pltpu.async_copy(rows.at[b], acc.at[sdbuf.at[d, 1]], ssem[b],
                             add=True)

        def swait(b, d):
            pltpu.make_async_copy(rows.at[b], acc.at[sdbuf.at[d, 1]],
                                  ssem[b]).wait()

        def icopy(j, d):
            pltpu.async_copy(sd_hbm.at[wid, j], sdbuf.at[d], isem[d])

        def iwait(j, d):
            pltpu.make_async_copy(sd_hbm.at[wid, 0], sdbuf.at[d],
                                  isem[d]).wait()

        def when_(cond, fn):
            if isinstance(cond, bool):
                if cond:
                    fn()
            else:
                pl.when(cond)(fn)

        icopy(0, 0)
        icopy(1, 1)
        iwait(0, 0)
        gather(0, 0, 0)
        _zero_stripe(zeros_hbm, acc, sid)
        plsc.subcore_barrier()

        def step(j, m):
            b = m % 4
            gwait(j, b, m)
            scat(j, b, m)

            def _next():
                when_(j >= 3,
                      lambda: swait((m + 1) % 4, (m + 5) % 8))
                iwait(j + 1, (m + 1) % 8)
                gather(j + 1, (b + 1) % 4, (m + 1) % 8)

            when_(j + 1 < NCH, _next)
            when_(j + 2 < NCH, lambda: icopy(j + 2, (m + 2) % 8))

        step(0, 0)
        step(1, 1)

        def oct_(t, carry):
            j0 = 8 * t + 2
            for i in range(8):
                step(j0 + i, (2 + i) % 8)
            return carry

        n_oct = (NCH - 2 - 3) // 8
        lax.fori_loop(0, n_oct, oct_, 0)
        for j in range(2 + 8 * n_oct, NCH):
            step(j, j % 8)
        swait((NCH - 4) % 4, (NCH - 4) % 8)
        swait((NCH - 3) % 4, (NCH - 3) % 8)
        swait((NCH - 2) % 4, (NCH - 2) % 8)
        swait((NCH - 1) % 4, (NCH - 1) % 8)
        plsc.subcore_barrier()
        _write_stripe(acc, out_hbm, cid, sid)

    return k(hs, sd4, zeros_d)


_RB = 2000


def _tc_proj(x, W, b):
    def body(x_ref, w_ref, b_ref, o_ref):
        h = jnp.dot(x_ref[...], w_ref[...], preferred_element_type=jnp.float32,
                    precision=lax.Precision.HIGHEST)
        o_ref[...] = jnp.maximum(h + b_ref[...][None, :], 0.0)

    return pl.pallas_call(
        body,
        grid=(N // _RB,),
        in_specs=[
            pl.BlockSpec((_RB, D), lambda i: (i, 0)),
            pl.BlockSpec((D, D), lambda i: (0, 0)),
            pl.BlockSpec((D,), lambda i: (0,)),
        ],
        out_specs=pl.BlockSpec((_RB, D), lambda i: (i, 0)),
        out_shape=jax.ShapeDtypeStruct((N, D), jnp.float32),
    )(x, W, b)


def _tc_hs0(h, W, dacc):
    def body(h_ref, w_ref, dacc_ref, hs_ref, dis_ref):
        deg = 1.0 + (dacc_ref[0, :, 0:1] + dacc_ref[1, :, 0:1])
        dis = lax.rsqrt(deg)
        dis_ref[...] = dis
        hl = jnp.dot(h_ref[...], w_ref[...], preferred_element_type=jnp.float32,
                     precision=lax.Precision.HIGHEST)
        hs_ref[...] = hl * dis

    return pl.pallas_call(
        body,
        grid=(N // _RB,),
        in_specs=[
            pl.BlockSpec((_RB, D), lambda i: (i, 0)),
            pl.BlockSpec((D, D), lambda i: (0, 0)),
            pl.BlockSpec((NC, _RB, DW), lambda i: (0, i, 0)),
        ],
        out_specs=[
            pl.BlockSpec((_RB, D), lambda i: (i, 0)),
            pl.BlockSpec((_RB, 1), lambda i: (i, 0)),
        ],
        out_shape=[
            jax.ShapeDtypeStruct((N, D), jnp.float32),
            jax.ShapeDtypeStruct((N, 1), jnp.float32),
        ],
    )(h, W, dacc)


def _tc_mid(h, hs, acc, dis, b, W_next):
    def body(h_ref, hs_ref, acc_ref, dis_ref, b_ref, w_ref, hn_ref, hsn_ref):
        dis = dis_ref[...]
        t = (acc_ref[0] + acc_ref[1] + hs_ref[...]) * dis
        h_new = h_ref[...] + jnp.maximum(t + b_ref[...][None, :], 0.0)
        hn_ref[...] = h_new
        hl = jnp.dot(h_new, w_ref[...], preferred_element_type=jnp.float32,
                     precision=lax.Precision.HIGHEST)
        hsn_ref[...] = hl * dis

    return pl.pallas_call(
        body,
        grid=(N // _RB,),
        in_specs=[
            pl.BlockSpec((_RB, D), lambda i: (i, 0)),
            pl.BlockSpec((_RB, D), lambda i: (i, 0)),
            pl.BlockSpec((NC, _RB, D), lambda i: (0, i, 0)),
            pl.BlockSpec((_RB, 1), lambda i: (i, 0)),
            pl.BlockSpec((D,), lambda i: (0,)),
            pl.BlockSpec((D, D), lambda i: (0, 0)),
        ],
        out_specs=[
            pl.BlockSpec((_RB, D), lambda i: (i, 0)),
            pl.BlockSpec((_RB, D), lambda i: (i, 0)),
        ],
        out_shape=[
            jax.ShapeDtypeStruct((N, D), jnp.float32),
            jax.ShapeDtypeStruct((N, D), jnp.float32),
        ],
    )(h, hs, acc, dis, b, W_next)


def _tc_fin(h, hs, acc, dis, b, W_out, b_out):
    def body(h_ref, hs_ref, acc_ref, dis_ref, b_ref, w_ref, bo_ref, o_ref):
        dis = dis_ref[...]
        t = (acc_ref[0] + acc_ref[1] + hs_ref[...]) * dis
        h_new = h_ref[...] + jnp.maximum(t + b_ref[...][None, :], 0.0)
        o = jnp.dot(h_new, w_ref[...], preferred_element_type=jnp.float32,
                    precision=lax.Precision.HIGHEST)
        o_ref[...] = o + bo_ref[...][None, :]

    return pl.pallas_call(
        body,
        grid=(N // _RB,),
        in_specs=[
            pl.BlockSpec((_RB, D), lambda i: (i, 0)),
            pl.BlockSpec((_RB, D), lambda i: (i, 0)),
            pl.BlockSpec((NC, _RB, D), lambda i: (0, i, 0)),
            pl.BlockSpec((_RB, 1), lambda i: (i, 0)),
            pl.BlockSpec((D,), lambda i: (0,)),
            pl.BlockSpec((D, D), lambda i: (0, 0)),
            pl.BlockSpec((D,), lambda i: (0,)),
        ],
        out_specs=pl.BlockSpec((_RB, D), lambda i: (i, 0)),
        out_shape=jax.ShapeDtypeStruct((N, D), jnp.float32),
    )(h, hs, acc, dis, b, W_out, b_out)


def kernel(x, edge_index, W_in, b_in, W_g0, b_g0, W_g1, b_g1, W_g2, b_g2,
           W_out, b_out):
    ei = edge_index.astype(jnp.int32)
    src3 = ei[0].reshape(NW, NCH, K)
    dst3 = ei[1].reshape(NW, NCH, K)
    sd4 = jnp.stack([src3, dst3], axis=2)
    ones_w = jnp.ones((K, DW), jnp.float32)
    zeros_w = jnp.zeros((N, DW), jnp.float32)
    zeros_d = jnp.zeros((N, D), jnp.float32)

    dacc = _sc_degree(dst3, ones_w, zeros_w)
    h = _tc_proj(x, W_in, b_in)
    hs, dis = _tc_hs0(h, W_g0, dacc)

    acc = _sc_edge_aggregate(hs, sd4, zeros_d)
    h, hs = _tc_mid(h, hs, acc, dis, b_g0, W_g1)

    acc = _sc_edge_aggregate(hs, sd4, zeros_d)
    h, hs = _tc_mid(h, hs, acc, dis, b_g1, W_g2)

    acc = _sc_edge_aggregate(hs, sd4, zeros_d)
    return _tc_fin(h, hs, acc, dis, b_g2, W_out, b_out)
